# baseline (speedup 1.0000x reference)
import functools

import jax
import jax.numpy as jnp
from jax import lax
from jax.experimental import pallas as pl
from jax.experimental.pallas import tpu as pltpu
from jax.experimental.pallas import tpu_sc as plsc

N = 10000
E = 320000

NC = 2
NS = 16
ROWB = 400
NRB = N // ROWB

RPS_A = 624
RPS_LAST = N - (NS - 1) * RPS_A


def _per_subcore_rows(s, copy):
    @pl.when(s < NS - 1)
    def _():
        copy(pl.multiple_of(s * RPS_A, 8), RPS_A)

    @pl.when(s == NS - 1)
    def _():
        copy((NS - 1) * RPS_A, RPS_LAST)


def _gelu(x):
    return 0.5 * x * (1.0 + lax.erf(x * 0.7071067811865476))


def _sc_mesh():
    return plsc.VectorSubcoreMesh(
        core_axis_name="c", subcore_axis_name="s", num_cores=NC, num_subcores=NS
    )


NB = 4
SEG = 25


def _edge_pipeline(h_hbm, acc_sp, row_hbm, col_hbm, base, NCH, K,
                   rowi, coli, rows, gsems, ssems):
    assert NCH % SEG == 0, (NCH, SEG)
    NSEG = NCH // SEG

    def gat(jj, b):
        return pltpu.make_async_copy(h_hbm.at[rowi.at[jj]], rows[b], gsems[b])

    def sca(jj, b):
        return pltpu.make_async_copy(rows[b], acc_sp.at[coli.at[jj]], ssems[b])

    def seg_body(g, carry):
        sb = base + g * SEG
        pltpu.sync_copy(row_hbm.at[pl.ds(sb, SEG)], rowi)
        pltpu.sync_copy(col_hbm.at[pl.ds(sb, SEG)], coli)

        for b in range(NB):
            gat(b, b).start()

        def rnd(i, c2):
            for b in range(NB):
                jj = i * NB + b

                @pl.when(jj < SEG)
                def _(jj=jj, b=b):
                    gat(jj, b).wait()
                    sca(jj, b).start(add=True)

            for b in range(NB):
                nxt = (i + 1) * NB + b

                @pl.when(nxt < SEG)
                def _(nxt=nxt, b=b):
                    sca(nxt - NB, b).wait()
                    gat(nxt, b).start()

            return c2

        lax.fori_loop(0, (SEG + NB - 1) // NB, rnd, 0)
        for b in range(NB):
            jlast = ((SEG - 1 - b) // NB) * NB + b
            if jlast >= max(0, SEG - NB):
                sca(jlast, b).wait()
        return carry

    lax.fori_loop(0, NSEG, seg_body, 0)


def _ring_scratch(K, HD):
    return [
        pltpu.VMEM_SHARED((N, HD), jnp.float32),
        pltpu.VMEM((SEG, K), jnp.int32),
        pltpu.VMEM((SEG, K), jnp.int32),
    ] + [pltpu.VMEM((K, HD), jnp.float32) for _ in range(NB)] \
      + [pltpu.SemaphoreType.DMA for _ in range(2 * NB)]


@functools.lru_cache(maxsize=None)
def _make_agg_fsplit(HD, K):
    NCH = E // (NS * K)

    @functools.partial(
        pl.kernel,
        out_type=jax.ShapeDtypeStruct((NC * N, HD), jnp.float32),
        mesh=_sc_mesh(),
        scratch_types=_ring_scratch(K, HD),
        compiler_params=pltpu.CompilerParams(use_tc_tiling_on_sc=False),
    )
    def agg(ha_hbm, hb_hbm, row_hbm, col_hbm, out_hbm, acc_sp, rowi, coli,
            *bufs):
        rows, sems = bufs[:NB], bufs[NB:]
        gsems, ssems = sems[:NB], sems[NB:]
        c = lax.axis_index("c")
        s = lax.axis_index("s")
        half = pl.multiple_of(c * N, 8)

        for ci, h_hbm in ((0, ha_hbm), (1, hb_hbm)):
            @pl.when(c == ci)
            def _(h_hbm=h_hbm):
                _per_subcore_rows(
                    s,
                    lambda st, nr: pltpu.sync_copy(
                        h_hbm.at[pl.ds(st, nr)], acc_sp.at[pl.ds(st, nr)]
                    ),
                )

        plsc.subcore_barrier()
        for ci, h_hbm in ((0, ha_hbm), (1, hb_hbm)):
            @pl.when(c == ci)
            def _(h_hbm=h_hbm):
                _edge_pipeline(h_hbm, acc_sp, row_hbm, col_hbm, s * NCH, NCH,
                               K, rowi, coli, rows, gsems, ssems)

        plsc.subcore_barrier()
        _per_subcore_rows(
            s,
            lambda st, nr: pltpu.sync_copy(
                acc_sp.at[pl.ds(st, nr)], out_hbm.at[pl.ds(half + st, nr)]
            ),
        )

    return agg


@functools.lru_cache(maxsize=None)
def _make_agg_esplit(D, K):
    NCH = E // (NC * NS * K)
    CPC = E // (NC * K)

    @functools.partial(
        pl.kernel,
        out_type=jax.ShapeDtypeStruct((NC * N, D), jnp.float32),
        mesh=_sc_mesh(),
        scratch_types=_ring_scratch(K, D),
        compiler_params=pltpu.CompilerParams(use_tc_tiling_on_sc=False),
    )
    def agg(h_hbm, z_hbm, row_hbm, col_hbm, out_hbm, acc_sp, rowi, coli,
            *bufs):
        rows, sems = bufs[:NB], bufs[NB:]
        gsems, ssems = sems[:NB], sems[NB:]
        c = lax.axis_index("c")
        s = lax.axis_index("s")
        cbase = c * CPC + s * NCH

        @pl.when(c == 0)
        def _():
            _per_subcore_rows(
                s,
                lambda st, nr: pltpu.sync_copy(
                    h_hbm.at[pl.ds(st, nr)], acc_sp.at[pl.ds(st, nr)]
                ),
            )

        @pl.when(c == 1)
        def _():
            _per_subcore_rows(
                s,
                lambda st, nr: pltpu.sync_copy(
                    z_hbm.at[pl.ds(st, nr)], acc_sp.at[pl.ds(st, nr)]
                ),
            )

        plsc.subcore_barrier()
        _edge_pipeline(h_hbm, acc_sp, row_hbm, col_hbm, cbase, NCH, K,
                       rowi, coli, rows, gsems, ssems)
        plsc.subcore_barrier()
        half = pl.multiple_of(c * N, 8)
        _per_subcore_rows(
            s,
            lambda st, nr: pltpu.sync_copy(
                acc_sp.at[pl.ds(st, nr)], out_hbm.at[pl.ds(half + st, nr)]
            ),
        )

    return agg


def _dotT(a, w):
    return lax.dot_general(
        a, w, (((1,), (1,)), ((), ())), preferred_element_type=jnp.float32
    )


def _pre_body(x_ref, deg_ref, out_ref):
    out_ref[...] = _gelu(x_ref[...]) / deg_ref[...]


def _pre(x, deg):
    return pl.pallas_call(
        _pre_body,
        grid=(NRB,),
        in_specs=[
            pl.BlockSpec((ROWB, 128), lambda r: (r, 0)),
            pl.BlockSpec((ROWB, 1), lambda r: (r, 0)),
        ],
        out_specs=pl.BlockSpec((ROWB, 128), lambda r: (r, 0)),
        out_shape=jax.ShapeDtypeStruct((N, 128), jnp.float32),
    )(x, deg)


def _mid0_body(xa_ref, xb_ref, wa_ref, wb_ref, b_ref, deg_ref,
               oa_ref, ob_ref):
    x = xa_ref[...] + xb_ref[...]
    rdeg = 1.0 / deg_ref[...]
    b = b_ref[...]
    oa_ref[...] = _gelu(_dotT(x, wa_ref[...]) + b[:, :128]) * rdeg
    ob_ref[...] = _gelu(_dotT(x, wb_ref[...]) + b[:, 128:]) * rdeg


def _mid0(parts, W, b, deg):
    return pl.pallas_call(
        _mid0_body,
        grid=(NRB,),
        in_specs=[
            pl.BlockSpec((ROWB, 128), lambda r: (r, 0)),
            pl.BlockSpec((ROWB, 128), lambda r: (NRB + r, 0)),
            pl.BlockSpec((128, 128), lambda r: (0, 0)),
            pl.BlockSpec((128, 128), lambda r: (1, 0)),
            pl.BlockSpec((1, 256), lambda r: (0, 0)),
            pl.BlockSpec((ROWB, 1), lambda r: (r, 0)),
        ],
        out_specs=[
            pl.BlockSpec((ROWB, 128), lambda r: (r, 0)),
            pl.BlockSpec((ROWB, 128), lambda r: (r, 0)),
        ],
        out_shape=[
            jax.ShapeDtypeStruct((N, 128), jnp.float32),
            jax.ShapeDtypeStruct((N, 128), jnp.float32),
        ],
    )(parts, parts, W, W, b.reshape(1, -1), deg)


def _mid1_body(xa_ref, xb_ref, w00, w01, w10, w11, b_ref, deg_ref,
               oa_ref, ob_ref):
    xa, xb = xa_ref[...], xb_ref[...]
    rdeg = 1.0 / deg_ref[...]
    b = b_ref[...]
    za = _dotT(xa, w00[...]) + _dotT(xb, w01[...]) + b[:, :128]
    zb = _dotT(xa, w10[...]) + _dotT(xb, w11[...]) + b[:, 128:]
    oa_ref[...] = _gelu(za) * rdeg
    ob_ref[...] = _gelu(zb) * rdeg


def _mid1(asplit, W, b, deg):
    wspec = lambda i, j: pl.BlockSpec((128, 128), lambda r, i=i, j=j: (i, j))
    return pl.pallas_call(
        _mid1_body,
        grid=(NRB,),
        in_specs=[
            pl.BlockSpec((ROWB, 128), lambda r: (r, 0)),
            pl.BlockSpec((ROWB, 128), lambda r: (NRB + r, 0)),
            wspec(0, 0), wspec(0, 1), wspec(1, 0), wspec(1, 1),
            pl.BlockSpec((1, 256), lambda r: (0, 0)),
            pl.BlockSpec((ROWB, 1), lambda r: (r, 0)),
        ],
        out_specs=[
            pl.BlockSpec((ROWB, 128), lambda r: (r, 0)),
            pl.BlockSpec((ROWB, 128), lambda r: (r, 0)),
        ],
        out_shape=[
            jax.ShapeDtypeStruct((N, 128), jnp.float32),
            jax.ShapeDtypeStruct((N, 128), jnp.float32),
        ],
    )(asplit, asplit, W, W, W, W, b.reshape(1, -1), deg)


def _tail_body(xa_ref, xb_ref, wm_a, wm_b, bm, wf0, bf0, wi0, bi0,
               wf1, bf1, wi1, bi1, wo, bo, out_ref):
    h3 = _dotT(xa_ref[...], wm_a[...]) + _dotT(xb_ref[...], wm_b[...]) + bm[...]
    t = _dotT(_gelu(h3), wf0[...]) + bf0[...] + _dotT(h3, wi0[...]) + bi0[...]
    t = _dotT(_gelu(t), wf1[...]) + bf1[...] + _dotT(h3, wi1[...]) + bi1[...]
    out_ref[...] = _dotT(t, wo[...]) + bo[...]


def _tail(asplit, W_mp2, b_mp2, W_fc0, b_fc0, W_fc1, b_fc1,
          W_inj0, b_inj0, W_inj1, b_inj1, W_out, b_out):
    full = lambda a, b: pl.BlockSpec((a, b), lambda r: (0, 0))
    return pl.pallas_call(
        _tail_body,
        grid=(NRB,),
        in_specs=[
            pl.BlockSpec((ROWB, 128), lambda r: (r, 0)),
            pl.BlockSpec((ROWB, 128), lambda r: (NRB + r, 0)),
            pl.BlockSpec((256, 128), lambda r: (0, 0)),
            pl.BlockSpec((256, 128), lambda r: (0, 1)),
            full(1, 256), full(256, 256), full(1, 256),
            full(256, 256), full(1, 256), full(256, 256), full(1, 256),
            full(256, 256), full(1, 256), full(128, 256), full(1, 128),
        ],
        out_specs=pl.BlockSpec((ROWB, 128), lambda r: (r, 0)),
        out_shape=jax.ShapeDtypeStruct((N, 128), jnp.float32),
    )(asplit, asplit, W_mp2, W_mp2, b_mp2.reshape(1, -1),
      W_fc0, b_fc0.reshape(1, -1), W_inj0, b_inj0.reshape(1, -1),
      W_fc1, b_fc1.reshape(1, -1), W_inj1, b_inj1.reshape(1, -1),
      W_out, b_out.reshape(1, -1))


def kernel(x, edge_index, W_mp0, b_mp0, W_mp1, b_mp1, W_mp2, b_mp2,
           W_fc0, b_fc0, W_fc1, b_fc1, W_inj0, b_inj0, W_inj1, b_inj1,
           W_out, b_out):
    K = 80
    row2d = edge_index[0].reshape(E // K, K)
    col2d = edge_index[1].reshape(E // K, K)

    ones16 = jnp.ones((N, 16), jnp.float32)
    zeros16 = jnp.zeros((N, 16), jnp.float32)
    degp = _make_agg_esplit(16, K)(ones16, zeros16, row2d, col2d)
    deg = degp[:N, :1] + degp[N:, :1]

    zeros128 = jnp.zeros((N, 128), jnp.float32)
    h0 = _pre(x, deg)
    a0 = _make_agg_esplit(128, K)(h0, zeros128, row2d, col2d)
    h1a, h1b = _mid0(a0, W_mp0, b_mp0, deg)
    a1 = _make_agg_fsplit(128, K)(h1a, h1b, row2d, col2d)
    h2a, h2b = _mid1(a1, W_mp1, b_mp1, deg)
    a2 = _make_agg_fsplit(128, K)(h2a, h2b, row2d, col2d)
    return _tail(a2, W_mp2, b_mp2, W_fc0, b_fc0, W_fc1, b_fc1,
                 W_inj0, b_inj0, W_inj1, b_inj1, W_out, b_out)


# speedup vs baseline: 1.0753x; 1.0753x over previous
import functools

import jax
import jax.numpy as jnp
from jax import lax
from jax.experimental import pallas as pl
from jax.experimental.pallas import tpu as pltpu
from jax.experimental.pallas import tpu_sc as plsc

N = 10000
E = 320000

NC = 2
NS = 16
ROWB = 400
NRB = N // ROWB

RPS_A = 624
RPS_LAST = N - (NS - 1) * RPS_A


def _per_subcore_rows(s, copy):
    @pl.when(s < NS - 1)
    def _():
        copy(pl.multiple_of(s * RPS_A, 8), RPS_A)

    @pl.when(s == NS - 1)
    def _():
        copy((NS - 1) * RPS_A, RPS_LAST)


def _gelu(x):
    return 0.5 * x * (1.0 + lax.erf(x * 0.7071067811865476))


def _sc_mesh():
    return plsc.VectorSubcoreMesh(
        core_axis_name="c", subcore_axis_name="s", num_cores=NC, num_subcores=NS
    )


NB = 4


def _edge_pipeline(h_hbm, acc_sp, row_hbm, col_hbm, base, NCH, K, SEG,
                   rowi, coli, rows, gsems, ssems):
    assert NCH % SEG == 0, (NCH, SEG)
    NSEG = NCH // SEG

    def gat(jj, b):
        return pltpu.make_async_copy(h_hbm.at[rowi.at[jj]], rows[b], gsems[b])

    def sca(jj, b):
        return pltpu.make_async_copy(rows[b], acc_sp.at[coli.at[jj]], ssems[b])

    def seg_body(g, carry):
        sb = base + g * SEG
        pltpu.sync_copy(row_hbm.at[pl.ds(sb, SEG)], rowi)
        pltpu.sync_copy(col_hbm.at[pl.ds(sb, SEG)], coli)

        for b in range(NB):
            gat(b, b).start()

        def rnd(i, c2):
            for b in range(NB):
                jj = i * NB + b

                @pl.when(jj < SEG)
                def _(jj=jj, b=b):
                    gat(jj, b).wait()
                    sca(jj, b).start(add=True)

            for b in range(NB):
                nxt = (i + 1) * NB + b

                @pl.when(nxt < SEG)
                def _(nxt=nxt, b=b):
                    sca(nxt - NB, b).wait()
                    gat(nxt, b).start()

            return c2

        lax.fori_loop(0, (SEG + NB - 1) // NB, rnd, 0)
        for b in range(NB):
            jlast = ((SEG - 1 - b) // NB) * NB + b
            if jlast >= max(0, SEG - NB):
                sca(jlast, b).wait()
        return carry

    lax.fori_loop(0, NSEG, seg_body, 0)


def _ring_scratch(K, HD, SEG):
    return [
        pltpu.VMEM_SHARED((N, HD), jnp.float32),
        pltpu.VMEM((SEG, K), jnp.int32),
        pltpu.VMEM((SEG, K), jnp.int32),
    ] + [pltpu.VMEM((K, HD), jnp.float32) for _ in range(NB)] \
      + [pltpu.SemaphoreType.DMA for _ in range(2 * NB)]


@functools.lru_cache(maxsize=None)
def _make_agg_fsplit(HD, K, SEG=50):
    NCH = E // (NS * K)

    @functools.partial(
        pl.kernel,
        out_type=jax.ShapeDtypeStruct((NC * N, HD), jnp.float32),
        mesh=_sc_mesh(),
        scratch_types=_ring_scratch(K, HD, SEG),
        compiler_params=pltpu.CompilerParams(use_tc_tiling_on_sc=False),
    )
    def agg(ha_hbm, hb_hbm, row_hbm, col_hbm, out_hbm, acc_sp, rowi, coli,
            *bufs):
        rows, sems = bufs[:NB], bufs[NB:]
        gsems, ssems = sems[:NB], sems[NB:]
        c = lax.axis_index("c")
        s = lax.axis_index("s")
        half = pl.multiple_of(c * N, 8)

        for ci, h_hbm in ((0, ha_hbm), (1, hb_hbm)):
            @pl.when(c == ci)
            def _(h_hbm=h_hbm):
                _per_subcore_rows(
                    s,
                    lambda st, nr: pltpu.sync_copy(
                        h_hbm.at[pl.ds(st, nr)], acc_sp.at[pl.ds(st, nr)]
                    ),
                )

        plsc.subcore_barrier()
        for ci, h_hbm in ((0, ha_hbm), (1, hb_hbm)):
            @pl.when(c == ci)
            def _(h_hbm=h_hbm):
                _edge_pipeline(h_hbm, acc_sp, row_hbm, col_hbm, s * NCH, NCH,
                               K, SEG, rowi, coli, rows, gsems, ssems)

        plsc.subcore_barrier()
        _per_subcore_rows(
            s,
            lambda st, nr: pltpu.sync_copy(
                acc_sp.at[pl.ds(st, nr)], out_hbm.at[pl.ds(half + st, nr)]
            ),
        )

    return agg


@functools.lru_cache(maxsize=None)
def _make_agg_esplit(D, K, SEG=25):
    NCH = E // (NC * NS * K)
    CPC = E // (NC * K)

    @functools.partial(
        pl.kernel,
        out_type=jax.ShapeDtypeStruct((NC * N, D), jnp.float32),
        mesh=_sc_mesh(),
        scratch_types=_ring_scratch(K, D, SEG),
        compiler_params=pltpu.CompilerParams(use_tc_tiling_on_sc=False),
    )
    def agg(h_hbm, z_hbm, row_hbm, col_hbm, out_hbm, acc_sp, rowi, coli,
            *bufs):
        rows, sems = bufs[:NB], bufs[NB:]
        gsems, ssems = sems[:NB], sems[NB:]
        c = lax.axis_index("c")
        s = lax.axis_index("s")
        cbase = c * CPC + s * NCH

        @pl.when(c == 0)
        def _():
            _per_subcore_rows(
                s,
                lambda st, nr: pltpu.sync_copy(
                    h_hbm.at[pl.ds(st, nr)], acc_sp.at[pl.ds(st, nr)]
                ),
            )

        @pl.when(c == 1)
        def _():
            _per_subcore_rows(
                s,
                lambda st, nr: pltpu.sync_copy(
                    z_hbm.at[pl.ds(st, nr)], acc_sp.at[pl.ds(st, nr)]
                ),
            )

        plsc.subcore_barrier()
        _edge_pipeline(h_hbm, acc_sp, row_hbm, col_hbm, cbase, NCH, K, SEG,
                       rowi, coli, rows, gsems, ssems)
        plsc.subcore_barrier()
        half = pl.multiple_of(c * N, 8)
        _per_subcore_rows(
            s,
            lambda st, nr: pltpu.sync_copy(
                acc_sp.at[pl.ds(st, nr)], out_hbm.at[pl.ds(half + st, nr)]
            ),
        )

    return agg


@functools.lru_cache(maxsize=None)
def _make_deg(K=125, SEGD=20, NBD=5):
    NCH = E // (NC * NS * K)
    CPC = E // (NC * K)
    assert NCH % SEGD == 0

    @functools.partial(
        pl.kernel,
        out_type=jax.ShapeDtypeStruct((NC * N, 16), jnp.float32),
        mesh=_sc_mesh(),
        scratch_types=[
            pltpu.VMEM_SHARED((N, 16), jnp.float32),
            pltpu.VMEM((SEGD, K), jnp.int32),
            pltpu.VMEM((K, 16), jnp.float32),
        ] + [pltpu.SemaphoreType.DMA for _ in range(NBD)],
        compiler_params=pltpu.CompilerParams(use_tc_tiling_on_sc=False),
    )
    def deg(ones_hbm, z_hbm, col_hbm, out_hbm, acc_sp, coli, ones_v, *ssems):
        c = lax.axis_index("c")
        s = lax.axis_index("s")

        def fill(i, carry):
            ones_v[i, :] = jnp.full((16,), 1.0, jnp.float32)
            return carry

        lax.fori_loop(0, K, fill, 0)

        for ci, src in ((0, ones_hbm), (1, z_hbm)):
            @pl.when(c == ci)
            def _(src=src):
                _per_subcore_rows(
                    s,
                    lambda st, nr: pltpu.sync_copy(
                        src.at[pl.ds(st, nr)], acc_sp.at[pl.ds(st, nr)]
                    ),
                )

        plsc.subcore_barrier()

        def sca(jj, b):
            return pltpu.make_async_copy(ones_v, acc_sp.at[coli.at[jj]],
                                         ssems[b])

        def seg_body(g, carry):
            pltpu.sync_copy(
                col_hbm.at[pl.ds(c * CPC + s * NCH + g * SEGD, SEGD)], coli
            )
            for b in range(NBD):
                sca(b, b).start(add=True)

            def rnd(i, c2):
                for b in range(NBD):
                    jj = i * NBD + b

                    @pl.when(jj < SEGD)
                    def _(jj=jj, b=b):
                        sca(jj - NBD, b).wait()
                        sca(jj, b).start(add=True)

                return c2

            lax.fori_loop(1, (SEGD + NBD - 1) // NBD, rnd, 0)
            for b in range(NBD):
                jlast = ((SEGD - 1 - b) // NBD) * NBD + b
                if jlast >= max(0, SEGD - NBD):
                    sca(jlast, b).wait()
            return carry

        lax.fori_loop(0, NCH // SEGD, seg_body, 0)
        plsc.subcore_barrier()
        half = pl.multiple_of(c * N, 8)
        _per_subcore_rows(
            s,
            lambda st, nr: pltpu.sync_copy(
                acc_sp.at[pl.ds(st, nr)], out_hbm.at[pl.ds(half + st, nr)]
            ),
        )

    return deg


def _dotT(a, w):
    return lax.dot_general(
        a, w, (((1,), (1,)), ((), ())), preferred_element_type=jnp.float32
    )


def _pre_body(x_ref, deg_ref, out_ref):
    out_ref[...] = _gelu(x_ref[...]) / deg_ref[...]


def _pre(x, deg):
    return pl.pallas_call(
        _pre_body,
        grid=(NRB,),
        in_specs=[
            pl.BlockSpec((ROWB, 128), lambda r: (r, 0)),
            pl.BlockSpec((ROWB, 1), lambda r: (r, 0)),
        ],
        out_specs=pl.BlockSpec((ROWB, 128), lambda r: (r, 0)),
        out_shape=jax.ShapeDtypeStruct((N, 128), jnp.float32),
    )(x, deg)


def _mid0_body(xa_ref, xb_ref, wa_ref, wb_ref, b_ref, deg_ref,
               oa_ref, ob_ref):
    x = xa_ref[...] + xb_ref[...]
    rdeg = 1.0 / deg_ref[...]
    b = b_ref[...]
    oa_ref[...] = _gelu(_dotT(x, wa_ref[...]) + b[:, :128]) * rdeg
    ob_ref[...] = _gelu(_dotT(x, wb_ref[...]) + b[:, 128:]) * rdeg


def _mid0(parts, W, b, deg):
    return pl.pallas_call(
        _mid0_body,
        grid=(NRB,),
        in_specs=[
            pl.BlockSpec((ROWB, 128), lambda r: (r, 0)),
            pl.BlockSpec((ROWB, 128), lambda r: (NRB + r, 0)),
            pl.BlockSpec((128, 128), lambda r: (0, 0)),
            pl.BlockSpec((128, 128), lambda r: (1, 0)),
            pl.BlockSpec((1, 256), lambda r: (0, 0)),
            pl.BlockSpec((ROWB, 1), lambda r: (r, 0)),
        ],
        out_specs=[
            pl.BlockSpec((ROWB, 128), lambda r: (r, 0)),
            pl.BlockSpec((ROWB, 128), lambda r: (r, 0)),
        ],
        out_shape=[
            jax.ShapeDtypeStruct((N, 128), jnp.float32),
            jax.ShapeDtypeStruct((N, 128), jnp.float32),
        ],
    )(parts, parts, W, W, b.reshape(1, -1), deg)


def _mid1_body(xa_ref, xb_ref, w00, w01, w10, w11, b_ref, deg_ref,
               oa_ref, ob_ref):
    xa, xb = xa_ref[...], xb_ref[...]
    rdeg = 1.0 / deg_ref[...]
    b = b_ref[...]
    za = _dotT(xa, w00[...]) + _dotT(xb, w01[...]) + b[:, :128]
    zb = _dotT(xa, w10[...]) + _dotT(xb, w11[...]) + b[:, 128:]
    oa_ref[...] = _gelu(za) * rdeg
    ob_ref[...] = _gelu(zb) * rdeg


def _mid1(asplit, W, b, deg):
    wspec = lambda i, j: pl.BlockSpec((128, 128), lambda r, i=i, j=j: (i, j))
    return pl.pallas_call(
        _mid1_body,
        grid=(NRB,),
        in_specs=[
            pl.BlockSpec((ROWB, 128), lambda r: (r, 0)),
            pl.BlockSpec((ROWB, 128), lambda r: (NRB + r, 0)),
            wspec(0, 0), wspec(0, 1), wspec(1, 0), wspec(1, 1),
            pl.BlockSpec((1, 256), lambda r: (0, 0)),
            pl.BlockSpec((ROWB, 1), lambda r: (r, 0)),
        ],
        out_specs=[
            pl.BlockSpec((ROWB, 128), lambda r: (r, 0)),
            pl.BlockSpec((ROWB, 128), lambda r: (r, 0)),
        ],
        out_shape=[
            jax.ShapeDtypeStruct((N, 128), jnp.float32),
            jax.ShapeDtypeStruct((N, 128), jnp.float32),
        ],
    )(asplit, asplit, W, W, W, W, b.reshape(1, -1), deg)


def _tail_body(xa_ref, xb_ref, wm_a, wm_b, bm, wf0, bf0, wi0, bi0,
               wf1, bf1, wi1, bi1, wo, bo, out_ref):
    h3 = _dotT(xa_ref[...], wm_a[...]) + _dotT(xb_ref[...], wm_b[...]) + bm[...]
    t = _dotT(_gelu(h3), wf0[...]) + bf0[...] + _dotT(h3, wi0[...]) + bi0[...]
    t = _dotT(_gelu(t), wf1[...]) + bf1[...] + _dotT(h3, wi1[...]) + bi1[...]
    out_ref[...] = _dotT(t, wo[...]) + bo[...]


def _tail(asplit, W_mp2, b_mp2, W_fc0, b_fc0, W_fc1, b_fc1,
          W_inj0, b_inj0, W_inj1, b_inj1, W_out, b_out):
    full = lambda a, b: pl.BlockSpec((a, b), lambda r: (0, 0))
    return pl.pallas_call(
        _tail_body,
        grid=(NRB,),
        in_specs=[
            pl.BlockSpec((ROWB, 128), lambda r: (r, 0)),
            pl.BlockSpec((ROWB, 128), lambda r: (NRB + r, 0)),
            pl.BlockSpec((256, 128), lambda r: (0, 0)),
            pl.BlockSpec((256, 128), lambda r: (0, 1)),
            full(1, 256), full(256, 256), full(1, 256),
            full(256, 256), full(1, 256), full(256, 256), full(1, 256),
            full(256, 256), full(1, 256), full(128, 256), full(1, 128),
        ],
        out_specs=pl.BlockSpec((ROWB, 128), lambda r: (r, 0)),
        out_shape=jax.ShapeDtypeStruct((N, 128), jnp.float32),
    )(asplit, asplit, W_mp2, W_mp2, b_mp2.reshape(1, -1),
      W_fc0, b_fc0.reshape(1, -1), W_inj0, b_inj0.reshape(1, -1),
      W_fc1, b_fc1.reshape(1, -1), W_inj1, b_inj1.reshape(1, -1),
      W_out, b_out.reshape(1, -1))


def kernel(x, edge_index, W_mp0, b_mp0, W_mp1, b_mp1, W_mp2, b_mp2,
           W_fc0, b_fc0, W_fc1, b_fc1, W_inj0, b_inj0, W_inj1, b_inj1,
           W_out, b_out):
    K = 80
    row2d = edge_index[0].reshape(E // K, K)
    col2d = edge_index[1].reshape(E // K, K)

    ones16 = jnp.ones((N, 16), jnp.float32)
    zeros16 = jnp.zeros((N, 16), jnp.float32)
    col2d_deg = edge_index[1].reshape(E // 125, 125)
    degp = _make_deg()(ones16, zeros16, col2d_deg)
    deg = degp[:N, :1] + degp[N:, :1]

    zeros128 = jnp.zeros((N, 128), jnp.float32)
    h0 = _pre(x, deg)
    a0 = _make_agg_esplit(128, K)(h0, zeros128, row2d, col2d)
    h1a, h1b = _mid0(a0, W_mp0, b_mp0, deg)
    a1 = _make_agg_fsplit(128, K)(h1a, h1b, row2d, col2d)
    h2a, h2b = _mid1(a1, W_mp1, b_mp1, deg)
    a2 = _make_agg_fsplit(128, K)(h2a, h2b, row2d, col2d)
    return _tail(a2, W_mp2, b_mp2, W_fc0, b_fc0, W_fc1, b_fc1,
                 W_inj0, b_inj0, W_inj1, b_inj1, W_out, b_out)
